# baseline (speedup 1.0000x reference)
import functools

import jax
import jax.numpy as jnp
from jax import lax
from jax.experimental import pallas as pl
from jax.experimental.pallas import tpu as pltpu
from jax.experimental.pallas import tpu_sc as plsc

_NTOKEN = 100000
_NINP = 300
_BATCH = 16384
_SEQ = 50

_NC = 2
_NS = 16
_NW = _NC * _NS

_DP = 304
_B = _BATCH * _SEQ
_BPW = _B // _NW
_CH = 128
_NCHUNKS = _BPW // _CH


@functools.partial(
    pl.kernel,
    mesh=plsc.VectorSubcoreMesh(core_axis_name="c", subcore_axis_name="s"),
    compiler_params=pltpu.CompilerParams(use_tc_tiling_on_sc=False),
    out_type=jax.ShapeDtypeStruct((_B, _DP), jnp.float32),
    scratch_types=[
        pltpu.VMEM((_CH,), jnp.int32),
        pltpu.VMEM((_CH, _DP), jnp.float32),
        pltpu.SemaphoreType.DMA,
    ],
)
def _gather_kernel(idx_hbm, table_hbm, out_hbm, idx_v, rows_v, sem):
    wid = lax.axis_index("s") * _NC + lax.axis_index("c")
    base = wid * _BPW

    def body(c, carry):
        start = base + c * _CH
        pltpu.sync_copy(idx_hbm.at[pl.ds(start, _CH)], idx_v)
        pltpu.async_copy(table_hbm.at[idx_v], rows_v, sem).wait()
        pltpu.sync_copy(rows_v, out_hbm.at[pl.ds(start, _CH)])
        return carry

    lax.fori_loop(0, _NCHUNKS, body, 0)


def kernel(input, table):
    idx = input.astype(jnp.int32).reshape(_B)
    table_p = jnp.pad(table, ((0, 0), (0, _DP - _NINP)))
    out = _gather_kernel(idx, table_p)
    return out[:, :_NINP].reshape(_BATCH, _SEQ, _NINP)


# speedup vs baseline: 1.0451x; 1.0451x over previous
import functools

import jax
import jax.numpy as jnp
from jax import lax
from jax.experimental import pallas as pl
from jax.experimental.pallas import tpu as pltpu
from jax.experimental.pallas import tpu_sc as plsc

_NTOKEN = 100000
_NINP = 300
_BATCH = 16384
_SEQ = 50

_NC = 2
_NS = 16
_NW = _NC * _NS

_DP = 304
_B = _BATCH * _SEQ
_BPW = _B // _NW
_CH = 128
_NCHUNKS = _BPW // _CH


@functools.partial(
    pl.kernel,
    mesh=plsc.VectorSubcoreMesh(core_axis_name="c", subcore_axis_name="s"),
    compiler_params=pltpu.CompilerParams(use_tc_tiling_on_sc=False),
    out_type=jax.ShapeDtypeStruct((_B, _DP), jnp.float32),
    scratch_types=[
        pltpu.VMEM((_NCHUNKS, _CH), jnp.int32),
        pltpu.VMEM((_CH, _DP), jnp.float32),
        pltpu.VMEM((_CH, _DP), jnp.float32),
        pltpu.SemaphoreType.DMA,
        pltpu.SemaphoreType.DMA,
        pltpu.SemaphoreType.DMA,
        pltpu.SemaphoreType.DMA,
    ],
)
def _gather_kernel(idx_hbm, table_hbm, out_hbm, idx_t, rows0, rows1,
                   sg0, sg1, sw0, sw1):
    wid = lax.axis_index("s") * _NC + lax.axis_index("c")
    base = wid * _BPW
    rows = (rows0, rows1)
    sg = (sg0, sg1)
    sw = (sw0, sw1)

    def g_start(c, b):
        pltpu.async_copy(table_hbm.at[idx_t.at[c]], rows[b], sg[b])

    def g_wait(b):
        pltpu.make_async_copy(table_hbm.at[idx_t.at[0]], rows[b],
                              sg[b]).wait()

    def w_start(c, b):
        pltpu.async_copy(rows[b], out_hbm.at[pl.ds(base + c * _CH, _CH)],
                         sw[b])

    def w_wait(b):
        pltpu.make_async_copy(rows[b], out_hbm.at[pl.ds(base, _CH)],
                              sw[b]).wait()

    pltpu.sync_copy(idx_hbm.at[wid], idx_t)
    g_start(0, 0)

    g_wait(0)
    w_start(0, 0)
    g_start(1, 1)

    def body(i, carry):
        for b, c in ((1, 2 * i - 1), (0, 2 * i)):
            g_wait(b)
            w_start(c, b)
            w_wait(1 - b)
            g_start(c + 1, 1 - b)
        return carry

    lax.fori_loop(1, _NCHUNKS // 2, body, 0)

    g_wait(1)
    w_start(_NCHUNKS - 1, 1)
    w_wait(0)
    w_wait(1)


def kernel(input, table):
    idx = input.astype(jnp.int32).reshape(_NW, _NCHUNKS, _CH)
    table_p = jnp.pad(table, ((0, 0), (0, _DP - _NINP)))
    out = _gather_kernel(idx, table_p)
    return out[:, :_NINP].reshape(_BATCH, _SEQ, _NINP)
